# Initial kernel scaffold; baseline (speedup 1.0000x reference)
import jax
import jax.numpy as jnp
from jax.experimental import pallas as pl


def kernel(x, emb_weight):
    raise NotImplementedError("write your pallas kernel here")


# speedup vs baseline: 1.5331x; 1.5331x over previous
import functools

import jax
import jax.numpy as jnp
from jax import lax
from jax.experimental import pallas as pl
from jax.experimental.pallas import tpu as pltpu
from jax.experimental.pallas import tpu_sc as plsc

_DIM = 1024
_ROWS = 8192
_NC, _NS = 2, 16
_NW = _NC * _NS
_ROWS_PER_W = _ROWS // _NW
_CHUNK = 32
_NCHUNK = _ROWS_PER_W // _CHUNK
_NBUF = 2


@functools.partial(
    pl.kernel,
    mesh=plsc.VectorSubcoreMesh(core_axis_name="c", subcore_axis_name="s"),
    out_type=jax.ShapeDtypeStruct((_ROWS, _DIM), jnp.float32),
    scratch_types=(
        [pltpu.VMEM((_CHUNK, _DIM), jnp.float32) for _ in range(_NBUF)]
        + [pltpu.SemaphoreType.DMA for _ in range(2 * _NBUF)]
    ),
)
def _sc_copy(emb_hbm, out_hbm, buf0, buf1, gsem0, gsem1, ssem0, ssem1):
    bufs = (buf0, buf1)
    gsems = (gsem0, gsem1)
    ssems = (ssem0, ssem1)
    wid = lax.axis_index("s") * _NC + lax.axis_index("c")
    base = wid * _ROWS_PER_W

    def gather(i):
        b = i % _NBUF
        return pltpu.make_async_copy(
            emb_hbm.at[pl.ds(base + i * _CHUNK, _CHUNK)], bufs[b], gsems[b])

    def scatter(i):
        b = i % _NBUF
        return pltpu.make_async_copy(
            bufs[b], out_hbm.at[pl.ds(base + i * _CHUNK, _CHUNK)], ssems[b])

    gather(0).start()
    for i in range(_NCHUNK):
        gather(i).wait()
        scatter(i).start()
        if i + 1 < _NCHUNK:
            if i >= 1:
                scatter(i - 1).wait()
            gather(i + 1).start()
    scatter(_NCHUNK - 2).wait()
    scatter(_NCHUNK - 1).wait()


def kernel(x, emb_weight):
    del x
    return _sc_copy(emb_weight)[None, :, :]
